# Initial kernel scaffold; baseline (speedup 1.0000x reference)
import jax
import jax.numpy as jnp
from jax.experimental import pallas as pl


def kernel(query, slot_keys, reliability_mask, W_router):
    raise NotImplementedError("write your pallas kernel here")


# speedup vs baseline: 1.7102x; 1.7102x over previous
import jax
import jax.numpy as jnp
import numpy as np
from jax.experimental import pallas as pl
from jax.experimental.pallas import tpu as pltpu

B, S, D = 4, 2048, 256
NUM_SLOTS = 32768
ROUTER_DIM = 48
K = 32
QT = 512
CT = 2048
NQ = (B * S) // QT
NC = NUM_SLOTS // CT
SCALE = np.float32(1.0 / np.sqrt(ROUTER_DIM))
NEG = np.float32(-np.inf)


def _project_kernel(q_ref, sk_ref, w_ref, rq_ref, rk_ref):
    w = w_ref[...]
    rq_ref[...] = jax.lax.dot_general(
        q_ref[...], w,
        dimension_numbers=(((1,), (1,)), ((), ())),
        preferred_element_type=jnp.float32,
    )
    rk_ref[...] = jax.lax.dot_general(
        sk_ref[...], w,
        dimension_numbers=(((1,), (1,)), ((), ())),
        preferred_element_type=jnp.float32,
    )


def _score_topk_kernel(rq_ref, rk_ref, mask_ref, vals_ref, idx_ref):
    rq = rq_ref[...]
    NCK = NC * K
    iota_ct = jax.lax.broadcasted_iota(jnp.int32, (QT, CT), 1)
    iota_k = jax.lax.broadcasted_iota(jnp.int32, (QT, K), 1)
    iota_nck = jax.lax.broadcasted_iota(jnp.int32, (QT, NCK), 1)

    def extract_body(j, ec):
        s_, v_, p_ = ec
        m = jnp.max(s_, axis=1, keepdims=True)
        i = jnp.min(jnp.where(s_ == m, iota_ct, CT), axis=1, keepdims=True)
        s_ = jnp.where(iota_ct == i, NEG, s_)
        v_ = jnp.where(iota_k == j, m, v_)
        p_ = jnp.where(iota_k == j, i, p_)
        return (s_, v_, p_)

    def chunk_body(c, carry):
        cv, cidx = carry
        rk_c = rk_ref[pl.ds(c * CT, CT), :]
        s = jax.lax.dot_general(
            rq, rk_c,
            dimension_numbers=(((1,), (1,)), ((), ())),
            preferred_element_type=jnp.float32,
        )
        s = s * SCALE + mask_ref[pl.ds(c, 1), :]
        v0 = jnp.full((QT, K), NEG, jnp.float32)
        p0 = jnp.zeros((QT, K), jnp.int32)
        _, v, p = jax.lax.fori_loop(0, K, extract_body, (s, v0, p0))
        vt = jnp.concatenate([v] * NC, axis=1)
        it = jnp.concatenate([p + c * CT] * NC, axis=1)
        blk = (iota_nck // K) == c
        return (jnp.where(blk, vt, cv), jnp.where(blk, it, cidx))

    cv0 = jnp.full((QT, NCK), NEG, jnp.float32)
    ci0 = jnp.zeros((QT, NCK), jnp.int32)
    cv, cidx = jax.lax.fori_loop(0, NC, chunk_body, (cv0, ci0))

    def merge_body(j, mc):
        cv_, vals_, io_ = mc
        m = jnp.max(cv_, axis=1, keepdims=True)
        ppos = jnp.min(jnp.where(cv_ == m, iota_nck, NCK), axis=1, keepdims=True)
        hit = iota_nck == ppos
        slot = jnp.sum(jnp.where(hit, cidx, 0), axis=1, keepdims=True)
        cv_ = jnp.where(hit, NEG, cv_)
        vals_ = jnp.where(iota_k == j, m, vals_)
        io_ = jnp.where(iota_k == j, slot, io_)
        return (cv_, vals_, io_)

    vals0 = jnp.full((QT, K), NEG, jnp.float32)
    io0 = jnp.zeros((QT, K), jnp.int32)
    _, vals, io = jax.lax.fori_loop(0, K, merge_body, (cv, vals0, io0))
    vals_ref[...] = vals
    idx_ref[...] = io


@jax.jit
def kernel(query, slot_keys, reliability_mask, W_router):
    q_flat = query.reshape(B * S, D)
    mask2d = reliability_mask.reshape(NC, CT)

    rq, rk = pl.pallas_call(
        _project_kernel,
        grid=(NC,),
        in_specs=[
            pl.BlockSpec((QT, D), lambda i: (i, 0)),
            pl.BlockSpec((CT, D), lambda i: (i, 0)),
            pl.BlockSpec((ROUTER_DIM, D), lambda i: (0, 0)),
        ],
        out_specs=[
            pl.BlockSpec((QT, ROUTER_DIM), lambda i: (i, 0)),
            pl.BlockSpec((CT, ROUTER_DIM), lambda i: (i, 0)),
        ],
        out_shape=[
            jax.ShapeDtypeStruct((B * S, ROUTER_DIM), jnp.float32),
            jax.ShapeDtypeStruct((NUM_SLOTS, ROUTER_DIM), jnp.float32),
        ],
        compiler_params=pltpu.CompilerParams(
            dimension_semantics=("parallel",),
        ),
    )(q_flat, slot_keys, W_router)

    top_vals, top_idx = pl.pallas_call(
        _score_topk_kernel,
        grid=(NQ,),
        in_specs=[
            pl.BlockSpec((QT, ROUTER_DIM), lambda qi: (qi, 0)),
            pl.BlockSpec((NUM_SLOTS, ROUTER_DIM), lambda qi: (0, 0)),
            pl.BlockSpec((NC, CT), lambda qi: (0, 0)),
        ],
        out_specs=[
            pl.BlockSpec((QT, K), lambda qi: (qi, 0)),
            pl.BlockSpec((QT, K), lambda qi: (qi, 0)),
        ],
        out_shape=[
            jax.ShapeDtypeStruct((B * S, K), jnp.float32),
            jax.ShapeDtypeStruct((B * S, K), jnp.int32),
        ],
        compiler_params=pltpu.CompilerParams(
            dimension_semantics=("parallel",),
        ),
    )(rq, rk, mask2d)

    return (top_idx.reshape(B, S, K), top_vals.reshape(B, S, K))
